# baseline (speedup 1.0000x reference)
import functools

import jax
import jax.numpy as jnp
from jax import lax
from jax.experimental import pallas as pl
from jax.experimental.pallas import tpu as pltpu
from jax.experimental.pallas import tpu_sc as plsc

N_IDX = 16384
NUM_CLASSES = 128
EMB_DIM = 9
N_TILES = 16
PER_TILE = N_IDX // N_TILES
LANES = 16
CHUNKS = NUM_CLASSES // LANES
TAB = NUM_CLASSES * EMB_DIM
LN2 = 0.6931471805599453


def _perm(x, idx):
    return x.at[idx].get(mode="promise_in_bounds")


def _bf_sum(x):
    lanes = jnp.arange(LANES, dtype=jnp.int32)
    for k in (1, 2, 4, 8):
        x = x + _perm(x, lanes ^ k)
    return x


def _bf_max(x):
    lanes = jnp.arange(LANES, dtype=jnp.int32)
    for k in (1, 2, 4, 8):
        x = jnp.maximum(x, _perm(x, lanes ^ k))
    return x


def _vlog(x):
    bits = lax.bitcast_convert_type(x, jnp.int32)
    e = ((bits >> 23) & 0xFF) - 127
    m = lax.bitcast_convert_type(
        (bits & 0x007FFFFF) | 0x3F800000, jnp.float32)
    t = m - 1.0
    y = e.astype(jnp.float32) * LN2 + t * (1.0 - t * (0.5 - t * (1.0 / 3.0)))
    for _ in range(3):
        y = y - 1.0 + x * jnp.exp(-y)
    return y


def _crow_body(idx_hbm, embf_hbm, wf_hbm, b_hbm, out_hbm,
               idx_v, loc_v, loc2_v, emb_v, wb_v, vec_v, pmall_v, lgall_v,
               out_v, sh_hist, sh_pm, sh_lg, idx_sem, emb_sem, wb_sem):
    wid = lax.axis_index("s")
    base = wid * PER_TILE

    idx_cp = pltpu.async_copy(idx_hbm.at[pl.ds(base, PER_TILE)], idx_v,
                              idx_sem)
    emb_cp = pltpu.async_copy(embf_hbm, emb_v, emb_sem)

    @pl.when(wid < CHUNKS)
    def _prefetch_wb():
        pltpu.async_copy(wf_hbm, wb_v.at[pl.ds(0, TAB)], wb_sem)
        pltpu.async_copy(b_hbm, wb_v.at[pl.ds(TAB, NUM_CLASSES)], wb_sem)

    zeros16 = jnp.zeros((LANES,), jnp.float32)
    for c in range(CHUNKS):
        loc_v[pl.ds(c * LANES, LANES)] = zeros16

    idx_cp.wait()

    ones16 = jnp.ones((LANES,), jnp.float32)
    for g in range(PER_TILE // LANES // 8):
        ivs = [idx_v[pl.ds((g * 8 + i) * LANES, LANES)] for i in range(8)]
        for iv in ivs:
            plsc.addupdate_scatter(loc_v, [iv], ones16)

    pltpu.sync_copy(loc_v, sh_hist.at[wid])
    pltpu.sync_copy(sh_hist.at[wid], loc2_v)

    emb_cp.wait()

    hist = [loc2_v[pl.ds(c * LANES, LANES)] for c in range(CHUNKS)]
    stride9 = jnp.arange(LANES, dtype=jnp.int32) * EMB_DIM
    lanes = jnp.arange(LANES, dtype=jnp.int32)
    pmv = zeros16
    for d in range(EMB_DIM):
        acc = zeros16
        for c in range(CHUNKS):
            col = plsc.load_gather(
                emb_v, [stride9 + (c * LANES * EMB_DIM + d)])
            acc = acc + hist[c] * col
        pmv = jnp.where(lanes == d, _bf_sum(acc), pmv)

    vec_v[pl.ds(0, LANES)] = pmv
    pltpu.sync_copy(vec_v, sh_pm.at[wid])
    plsc.subcore_barrier()

    @pl.when(wid < CHUNKS)
    def _logit_chunk():
        pltpu.make_async_copy(wf_hbm, wb_v.at[pl.ds(0, TAB)], wb_sem).wait()
        pltpu.make_async_copy(b_hbm, wb_v.at[pl.ds(TAB, NUM_CLASSES)],
                              wb_sem).wait()
        pltpu.sync_copy(sh_pm, pmall_v)

        msum = zeros16
        for w in range(N_TILES):
            msum = msum + pmall_v[w, pl.ds(0, LANES)]
        msum = msum * (1.0 / N_IDX)

        mvecs = [_perm(msum, jnp.full((LANES,), d, jnp.int32))
                 for d in range(EMB_DIM)]

        coff = wid * LANES * EMB_DIM
        acc = plsc.load_gather(wb_v, [lanes + (TAB + wid * LANES)])
        for d in range(EMB_DIM):
            wcol = plsc.load_gather(wb_v, [stride9 + coff + d])
            acc = acc + mvecs[d] * wcol

        vec_v[pl.ds(0, LANES)] = acc
        pltpu.sync_copy(vec_v, sh_lg.at[wid])

    plsc.subcore_barrier()

    @pl.when(wid == 0)
    def _finish():
        pltpu.sync_copy(sh_lg, lgall_v)
        logits = [lgall_v[w, pl.ds(0, LANES)] for w in range(CHUNKS)]

        m16 = logits[0]
        for c in range(1, CHUNKS):
            m16 = jnp.maximum(m16, logits[c])
        mx = _bf_max(m16)

        es = zeros16
        for c in range(CHUNKS):
            es = es + jnp.exp(logits[c] - mx)
        lse = _vlog(_bf_sum(es))

        for c in range(CHUNKS):
            out_v[pl.ds(c * LANES, LANES)] = logits[c] - mx - lse
        pltpu.sync_copy(out_v, out_hbm.at[0])


@jax.jit
def _crow(idx, emb_flat, w_flat, b):
    mesh = plsc.VectorSubcoreMesh(
        core_axis_name="c", subcore_axis_name="s", num_cores=1)
    f = functools.partial(
        pl.kernel,
        mesh=mesh,
        out_type=jax.ShapeDtypeStruct((1, NUM_CLASSES), jnp.float32),
        scratch_types=[
            pltpu.VMEM((PER_TILE,), jnp.int32),
            pltpu.VMEM((NUM_CLASSES,), jnp.float32),
            pltpu.VMEM((NUM_CLASSES,), jnp.float32),
            pltpu.VMEM((TAB,), jnp.float32),
            pltpu.VMEM((TAB + NUM_CLASSES,), jnp.float32),
            pltpu.VMEM((LANES,), jnp.float32),
            pltpu.VMEM((N_TILES, LANES), jnp.float32),
            pltpu.VMEM((CHUNKS, LANES), jnp.float32),
            pltpu.VMEM((NUM_CLASSES,), jnp.float32),
            pltpu.VMEM_SHARED((N_TILES, NUM_CLASSES), jnp.float32),
            pltpu.VMEM_SHARED((N_TILES, LANES), jnp.float32),
            pltpu.VMEM_SHARED((CHUNKS, LANES), jnp.float32),
            pltpu.SemaphoreType.DMA,
            pltpu.SemaphoreType.DMA,
            pltpu.SemaphoreType.DMA,
        ],
        compiler_params=pltpu.CompilerParams(needs_layout_passes=False),
    )(_crow_body)
    return f(idx, emb_flat, w_flat, b)


def kernel(inputs, emb_table, W, b):
    idx = inputs.astype(jnp.int32)
    emb_flat = emb_table.reshape(-1)
    w_flat = W.reshape(-1)
    return _crow(idx, emb_flat, w_flat, b)


# speedup vs baseline: 1.0249x; 1.0249x over previous
import functools

import jax
import jax.numpy as jnp
from jax import lax
from jax.experimental import pallas as pl
from jax.experimental.pallas import tpu as pltpu
from jax.experimental.pallas import tpu_sc as plsc

N_IDX = 16384
NUM_CLASSES = 128
EMB_DIM = 9
N_TILES = 16
PER_TILE = N_IDX // N_TILES
LANES = 16
CHUNKS = NUM_CLASSES // LANES
TAB = NUM_CLASSES * EMB_DIM
LN2 = 0.6931471805599453


def _perm(x, idx):
    return x.at[idx].get(mode="promise_in_bounds")


def _bf_sum(x):
    lanes = jnp.arange(LANES, dtype=jnp.int32)
    for k in (1, 2, 4, 8):
        x = x + _perm(x, lanes ^ k)
    return x


def _bf_max(x):
    lanes = jnp.arange(LANES, dtype=jnp.int32)
    for k in (1, 2, 4, 8):
        x = jnp.maximum(x, _perm(x, lanes ^ k))
    return x


def _vlog(x):
    bits = lax.bitcast_convert_type(x, jnp.int32)
    e = ((bits >> 23) & 0xFF) - 127
    m = lax.bitcast_convert_type(
        (bits & 0x007FFFFF) | 0x3F800000, jnp.float32)
    t = m - 1.0
    y = e.astype(jnp.float32) * LN2 + t * (1.0 - t * (0.5 - t * (1.0 / 3.0)))
    for _ in range(3):
        y = y - 1.0 + x * jnp.exp(-y)
    return y


def _crow_body(idx_hbm, embf_hbm, wf_hbm, b_hbm, out_hbm,
               idx_v, loc_v, allh_v, tab_v, out_v, shared_h,
               idx_sem, tab_sem):
    wid = lax.axis_index("s")
    base = wid * PER_TILE

    idx_cp = pltpu.async_copy(idx_hbm.at[pl.ds(base, PER_TILE)], idx_v,
                              idx_sem)

    @pl.when(wid == 0)
    def _prefetch_tables():
        pltpu.async_copy(embf_hbm, tab_v.at[pl.ds(0, TAB)], tab_sem)
        pltpu.async_copy(wf_hbm, tab_v.at[pl.ds(TAB, TAB)], tab_sem)
        pltpu.async_copy(b_hbm, tab_v.at[pl.ds(2 * TAB, NUM_CLASSES)],
                         tab_sem)

    zeros16 = jnp.zeros((LANES,), jnp.float32)
    for c in range(CHUNKS):
        loc_v[pl.ds(c * LANES, LANES)] = zeros16

    idx_cp.wait()

    ones16 = jnp.ones((LANES,), jnp.float32)
    for g in range(PER_TILE // LANES // 8):
        ivs = [idx_v[pl.ds((g * 8 + i) * LANES, LANES)] for i in range(8)]
        for iv in ivs:
            plsc.addupdate_scatter(loc_v, [iv], ones16)

    pltpu.sync_copy(loc_v, shared_h.at[wid])
    plsc.subcore_barrier()

    @pl.when(wid == 0)
    def _tail():
        pltpu.sync_copy(shared_h, allh_v)
        pltpu.make_async_copy(embf_hbm, tab_v.at[pl.ds(0, TAB)],
                              tab_sem).wait()
        pltpu.make_async_copy(wf_hbm, tab_v.at[pl.ds(TAB, TAB)],
                              tab_sem).wait()
        pltpu.make_async_copy(b_hbm, tab_v.at[pl.ds(2 * TAB, NUM_CLASSES)],
                              tab_sem).wait()

        hist = []
        for c in range(CHUNKS):
            acc = zeros16
            for w in range(N_TILES):
                acc = acc + allh_v[w, pl.ds(c * LANES, LANES)]
            hist.append(acc)

        stride9 = jnp.arange(LANES, dtype=jnp.int32) * EMB_DIM
        mean = []
        for d in range(EMB_DIM):
            acc = zeros16
            for c in range(CHUNKS):
                col = plsc.load_gather(
                    tab_v, [stride9 + (c * LANES * EMB_DIM + d)])
                acc = acc + hist[c] * col
            mean.append(_bf_sum(acc) * (1.0 / N_IDX))

        logits = []
        for c in range(CHUNKS):
            acc = tab_v[pl.ds(2 * TAB + c * LANES, LANES)]
            for d in range(EMB_DIM):
                wcol = plsc.load_gather(
                    tab_v, [stride9 + (TAB + c * LANES * EMB_DIM + d)])
                acc = acc + mean[d] * wcol
            logits.append(acc)

        m16 = logits[0]
        for c in range(1, CHUNKS):
            m16 = jnp.maximum(m16, logits[c])
        mx = _bf_max(m16)

        es = zeros16
        for c in range(CHUNKS):
            es = es + jnp.exp(logits[c] - mx)
        lse = _vlog(_bf_sum(es))

        for c in range(CHUNKS):
            out_v[pl.ds(c * LANES, LANES)] = logits[c] - mx - lse
        pltpu.sync_copy(out_v, out_hbm.at[0])


@jax.jit
def _crow(idx, emb_flat, w_flat, b):
    mesh = plsc.VectorSubcoreMesh(
        core_axis_name="c", subcore_axis_name="s", num_cores=1)
    f = functools.partial(
        pl.kernel,
        mesh=mesh,
        out_type=jax.ShapeDtypeStruct((1, NUM_CLASSES), jnp.float32),
        scratch_types=[
            pltpu.VMEM((PER_TILE,), jnp.int32),
            pltpu.VMEM((NUM_CLASSES,), jnp.float32),
            pltpu.VMEM((N_TILES, NUM_CLASSES), jnp.float32),
            pltpu.VMEM((2 * TAB + NUM_CLASSES,), jnp.float32),
            pltpu.VMEM((NUM_CLASSES,), jnp.float32),
            pltpu.VMEM_SHARED((N_TILES, NUM_CLASSES), jnp.float32),
            pltpu.SemaphoreType.DMA,
            pltpu.SemaphoreType.DMA,
        ],
        compiler_params=pltpu.CompilerParams(needs_layout_passes=False),
    )(_crow_body)
    return f(idx, emb_flat, w_flat, b)


def kernel(inputs, emb_table, W, b):
    idx = inputs.astype(jnp.int32)
    emb_flat = emb_table.reshape(-1)
    w_flat = W.reshape(-1)
    return _crow(idx, emb_flat, w_flat, b)
